# baseline (speedup 1.0000x reference)
import functools

import jax
import jax.numpy as jnp
from jax import lax
from jax.experimental import pallas as pl
from jax.experimental.pallas import tpu as pltpu
from jax.experimental.pallas import tpu_sc as plsc

B = 128
N = 32768
A = 4
NC = 2
NS = 16
NW = NC * NS
BPW = B // NW
L = 16
GRP = 512
NG = N // GRP
VPG = GRP // L
CHK = 8192
CPB = N // CHK
GPC = CHK // GRP
STEPS = BPW * CPB
BIG = 1 << 20


def _sreduce(vec, init, op):
    acc = init
    for i in range(L):
        acc = op(acc, vec[i])
    return acc


def _planner_body(ev_hbm, act_hbm, out_hbm, buf0_v, buf1_v, buf2_v, buf3_v,
                  cbuf_v, gbuf_v, obuf_v, mg_v, gs_v,
                  sem0, sem1, sem2, sem3, csem, gsem):
    wid = lax.axis_index("s") * NC + lax.axis_index("c")
    iota = lax.iota(jnp.int32, L)
    bufs = (buf0_v, buf1_v, buf2_v, buf3_v)
    sems = (sem0, sem1, sem2, sem3)
    b0 = wid * BPW

    def start(c, p):
        src = (b0 + lax.shift_right_logical(c, 2)) * N \
            + lax.bitwise_and(c, CPB - 1) * CHK
        pltpu.async_copy(
            ev_hbm.at[pl.ds(pl.multiple_of(src, L), CHK)], bufs[p], sems[p])

    for pp in range(4):
        start(jnp.int32(pp), pp)

    def super_body(s, carry):
        macc, gacc = carry
        for p in range(4):
            c = s * 4 + p
            pltpu.make_async_copy(
                ev_hbm.at[pl.ds(0, CHK)], bufs[p], sems[p]).wait()
            buf = bufs[p]

            def group_body(j, carry, buf=buf, c=c):
                macc, gacc = carry
                base = j * GRP
                m0 = buf[pl.ds(base, L)]
                m1 = buf[pl.ds(base + L, L)]
                m2 = buf[pl.ds(base + 2 * L, L)]
                m3 = buf[pl.ds(base + 3 * L, L)]
                for t in range(4, VPG, 4):
                    m0 = jnp.maximum(m0, buf[pl.ds(base + t * L, L)])
                    m1 = jnp.maximum(m1, buf[pl.ds(base + (t + 1) * L, L)])
                    m2 = jnp.maximum(m2, buf[pl.ds(base + (t + 2) * L, L)])
                    m3 = jnp.maximum(m3, buf[pl.ds(base + (t + 3) * L, L)])
                mm = jnp.maximum(jnp.maximum(m0, m1), jnp.maximum(m2, m3))
                upd = mm > macc
                g = lax.bitwise_and(c, CPB - 1) * GPC + j
                return jnp.where(upd, mm, macc), jnp.where(upd, g, gacc)

            macc, gacc = lax.fori_loop(0, GPC, group_body, (macc, gacc))

            @pl.when(c + 4 < STEPS)
            def _(c=c, p=p):
                start(c + 4, p)

            boundary = lax.bitwise_and(c, CPB - 1) == CPB - 1

            @pl.when(boundary)
            def _(c=c, macc=macc, gacc=gacc):
                rr = lax.shift_right_logical(c, 2)
                m = _sreduce(macc, jnp.float32(-jnp.inf), jnp.maximum)
                gcand = jnp.where(macc == m, gacc, BIG)
                gstar = _sreduce(gcand, jnp.int32(BIG), jnp.minimum)
                mg_v[pl.ds(rr * L, L)] = m + jnp.zeros((L,), jnp.float32)
                gs_v[pl.ds(rr * L, L)] = gstar + jnp.zeros((L,), jnp.int32)
                src = (b0 + rr) * N + gstar * GRP
                pltpu.async_copy(
                    ev_hbm.at[pl.ds(pl.multiple_of(src, L), GRP)],
                    cbuf_v.at[pl.ds(rr * GRP, GRP)], csem)

            macc = jnp.where(boundary, jnp.float32(-jnp.inf), macc)
            gacc = jnp.where(boundary, jnp.int32(BIG), gacc)
        return macc, gacc

    lax.fori_loop(
        0, STEPS // 4, super_body,
        (jnp.full((L,), -jnp.inf, jnp.float32),
         jnp.full((L,), BIG, jnp.int32)))

    pltpu.make_async_copy(
        ev_hbm.at[pl.ds(0, BPW * GRP)], cbuf_v, csem).wait()
    idxs = []
    for r in range(BPW):
        mv = mg_v[pl.ds(r * L, L)]

        def cfind_body(j, acc, r=r, mv=mv):
            v = cbuf_v[pl.ds(r * GRP + j * L, L)]
            return jnp.where(v == mv, jnp.minimum(acc, j), acc)

        jacc = lax.fori_loop(
            0, VPG, cfind_body, jnp.full((L,), BIG, jnp.int32))
        rel = _sreduce(jacc * L + iota, jnp.int32(BIG * L * 2), jnp.minimum)
        idx = gs_v[pl.ds(r * L, L)][0] * GRP + rel
        idxs.append(idx)
        grp = (b0 + r) * (N * A) + lax.shift_right_logical(idx, 7) * 512
        pltpu.async_copy(act_hbm.at[pl.ds(pl.multiple_of(grp, L), 512)],
                         gbuf_v.at[pl.ds(r * 512, 512)], gsem)

    pltpu.make_async_copy(
        act_hbm.at[pl.ds(0, BPW * 512)], gbuf_v, gsem).wait()
    outv = jnp.zeros((L,), jnp.float32)
    for r in range(BPW):
        off = lax.bitwise_and(idxs[r], 127)
        o8 = lax.bitwise_and(off, 7)
        base8 = r * 512 + off - o8
        for a in range(A):
            va = gbuf_v[pl.ds(pl.multiple_of(base8 + a * 128, 8), L)]
            sa = va[7]
            for i in range(6, -1, -1):
                sa = jnp.where(o8 == i, va[i], sa)
            outv = jnp.where(iota == r * A + a, sa, outv)
    obuf_v[...] = outv
    pltpu.sync_copy(
        obuf_v, out_hbm.at[pl.ds(pl.multiple_of(wid * L, L), L)])


_planner = functools.partial(
    pl.kernel,
    out_type=jax.ShapeDtypeStruct((B * A,), jnp.float32),
    mesh=plsc.VectorSubcoreMesh(core_axis_name="c", subcore_axis_name="s"),
    scratch_types=[
        pltpu.VMEM((CHK,), jnp.float32),
        pltpu.VMEM((CHK,), jnp.float32),
        pltpu.VMEM((CHK,), jnp.float32),
        pltpu.VMEM((CHK,), jnp.float32),
        pltpu.VMEM((BPW * GRP,), jnp.float32),
        pltpu.VMEM((BPW * 512,), jnp.float32),
        pltpu.VMEM((L,), jnp.float32),
        pltpu.VMEM((BPW * L,), jnp.float32),
        pltpu.VMEM((BPW * L,), jnp.int32),
        pltpu.SemaphoreType.DMA,
        pltpu.SemaphoreType.DMA,
        pltpu.SemaphoreType.DMA,
        pltpu.SemaphoreType.DMA,
        pltpu.SemaphoreType.DMA,
        pltpu.SemaphoreType.DMA,
    ],
)(_planner_body)


def kernel(estimated_value, action):
    ev = estimated_value.reshape(B * N)
    act = action.reshape(B, N // 128, 128, A)
    act = act.transpose(0, 1, 3, 2).reshape(B * N * A)
    out = _planner(ev, act)
    return out.reshape(B, A)


# speedup vs baseline: 1.0093x; 1.0093x over previous
import functools

import jax
import jax.numpy as jnp
from jax import lax
from jax.experimental import pallas as pl
from jax.experimental.pallas import tpu as pltpu
from jax.experimental.pallas import tpu_sc as plsc

B = 128
N = 32768
A = 4
NC = 2
NS = 16
NW = NC * NS
BPW = B // NW
L = 16
GRP = 512
NG = N // GRP
VPG = GRP // L
CHK = 8192
CPB = N // CHK
GPC = CHK // GRP
STEPS = BPW * CPB
BIG = 1 << 20


def _sreduce(vec, init, op):
    acc = init
    for i in range(L):
        acc = op(acc, vec[i])
    return acc


def _planner_body(ev_hbm, act_hbm, out_hbm, buf0_v, buf1_v, buf2_v, buf3_v,
                  cbuf_v, gbuf_v, obuf_v, mg_v, gs_v,
                  sem0, sem1, sem2, sem3, csem, gsem):
    wid = lax.axis_index("s") * NC + lax.axis_index("c")
    iota = lax.iota(jnp.int32, L)
    bufs = (buf0_v, buf1_v, buf2_v, buf3_v)
    sems = (sem0, sem1, sem2, sem3)
    b0 = wid * BPW

    def start(c, p):
        src = (b0 + lax.shift_right_logical(c, 2)) * N \
            + lax.bitwise_and(c, CPB - 1) * CHK
        pltpu.async_copy(
            ev_hbm.at[pl.ds(pl.multiple_of(src, L), CHK)], bufs[p], sems[p])

    for pp in range(4):
        start(jnp.int32(pp), pp)

    def super_body(s, carry):
        macc, gacc = carry
        for p in range(4):
            c = s * 4 + p
            pltpu.make_async_copy(
                ev_hbm.at[pl.ds(0, CHK)], bufs[p], sems[p]).wait()
            buf = bufs[p]

            def group_body(j, carry, buf=buf, c=c):
                macc, gacc = carry
                base = j * GRP
                m0 = buf[pl.ds(base, L)]
                m1 = buf[pl.ds(base + L, L)]
                m2 = buf[pl.ds(base + 2 * L, L)]
                m3 = buf[pl.ds(base + 3 * L, L)]
                for t in range(4, VPG, 4):
                    m0 = jnp.maximum(m0, buf[pl.ds(base + t * L, L)])
                    m1 = jnp.maximum(m1, buf[pl.ds(base + (t + 1) * L, L)])
                    m2 = jnp.maximum(m2, buf[pl.ds(base + (t + 2) * L, L)])
                    m3 = jnp.maximum(m3, buf[pl.ds(base + (t + 3) * L, L)])
                mm = jnp.maximum(jnp.maximum(m0, m1), jnp.maximum(m2, m3))
                upd = mm > macc
                g = lax.bitwise_and(c, CPB - 1) * GPC + j
                return jnp.where(upd, mm, macc), jnp.where(upd, g, gacc)

            macc, gacc = lax.fori_loop(0, GPC, group_body, (macc, gacc))

            @pl.when(c + 4 < STEPS)
            def _(c=c, p=p):
                start(c + 4, p)

            boundary = lax.bitwise_and(c, CPB - 1) == CPB - 1

            @pl.when(boundary)
            def _(c=c, macc=macc, gacc=gacc):
                rr = lax.shift_right_logical(c, 2)
                m = _sreduce(macc, jnp.float32(-jnp.inf), jnp.maximum)
                gcand = jnp.where(macc == m, gacc, BIG)
                gstar = _sreduce(gcand, jnp.int32(BIG), jnp.minimum)
                mg_v[pl.ds(rr * L, L)] = m + jnp.zeros((L,), jnp.float32)
                gs_v[pl.ds(rr * L, L)] = gstar + jnp.zeros((L,), jnp.int32)
                src = (b0 + rr) * N + gstar * GRP
                pltpu.async_copy(
                    ev_hbm.at[pl.ds(pl.multiple_of(src, L), GRP)],
                    cbuf_v.at[pl.ds(rr * GRP, GRP)], csem)

            macc = jnp.where(boundary, jnp.float32(-jnp.inf), macc)
            gacc = jnp.where(boundary, jnp.int32(BIG), gacc)
        return macc, gacc

    lax.fori_loop(
        0, STEPS // 4, super_body,
        (jnp.full((L,), -jnp.inf, jnp.float32),
         jnp.full((L,), BIG, jnp.int32)))

    pltpu.make_async_copy(
        ev_hbm.at[pl.ds(0, BPW * GRP)], cbuf_v, csem).wait()
    idxs = []
    for r in range(BPW):
        mv = mg_v[pl.ds(r * L, L)]

        def cfind_body(q, acc, r=r, mv=mv):
            for u in range(4):
                j = q * 4 + u
                v = cbuf_v[pl.ds(r * GRP + j * L, L)]
                acc = jnp.where(v == mv, jnp.minimum(acc, j), acc)
            return acc

        jacc = lax.fori_loop(
            0, VPG // 4, cfind_body, jnp.full((L,), BIG, jnp.int32))
        rel = _sreduce(jacc * L + iota, jnp.int32(BIG * L * 2), jnp.minimum)
        idx = gs_v[pl.ds(r * L, L)][0] * GRP + rel
        idxs.append(idx)
        grp = (b0 + r) * (N * A) + lax.shift_right_logical(idx, 7) * 512
        pltpu.async_copy(act_hbm.at[pl.ds(pl.multiple_of(grp, L), 512)],
                         gbuf_v.at[pl.ds(r * 512, 512)], gsem)

    pltpu.make_async_copy(
        act_hbm.at[pl.ds(0, BPW * 512)], gbuf_v, gsem).wait()
    outv = jnp.zeros((L,), jnp.float32)
    for r in range(BPW):
        off = lax.bitwise_and(idxs[r], 127)
        o8 = lax.bitwise_and(off, 7)
        base8 = r * 512 + off - o8
        for a in range(A):
            va = gbuf_v[pl.ds(pl.multiple_of(base8 + a * 128, 8), L)]
            sa = va[7]
            for i in range(6, -1, -1):
                sa = jnp.where(o8 == i, va[i], sa)
            outv = jnp.where(iota == r * A + a, sa, outv)
    obuf_v[...] = outv
    pltpu.sync_copy(
        obuf_v, out_hbm.at[pl.ds(pl.multiple_of(wid * L, L), L)])


_planner = functools.partial(
    pl.kernel,
    out_type=jax.ShapeDtypeStruct((B * A,), jnp.float32),
    mesh=plsc.VectorSubcoreMesh(core_axis_name="c", subcore_axis_name="s"),
    scratch_types=[
        pltpu.VMEM((CHK,), jnp.float32),
        pltpu.VMEM((CHK,), jnp.float32),
        pltpu.VMEM((CHK,), jnp.float32),
        pltpu.VMEM((CHK,), jnp.float32),
        pltpu.VMEM((BPW * GRP,), jnp.float32),
        pltpu.VMEM((BPW * 512,), jnp.float32),
        pltpu.VMEM((L,), jnp.float32),
        pltpu.VMEM((BPW * L,), jnp.float32),
        pltpu.VMEM((BPW * L,), jnp.int32),
        pltpu.SemaphoreType.DMA,
        pltpu.SemaphoreType.DMA,
        pltpu.SemaphoreType.DMA,
        pltpu.SemaphoreType.DMA,
        pltpu.SemaphoreType.DMA,
        pltpu.SemaphoreType.DMA,
    ],
)(_planner_body)


def kernel(estimated_value, action):
    ev = estimated_value.reshape(B * N)
    act = action.reshape(B, N // 128, 128, A)
    act = act.transpose(0, 1, 3, 2).reshape(B * N * A)
    out = _planner(ev, act)
    return out.reshape(B, A)
